# Initial kernel scaffold; baseline (speedup 1.0000x reference)
import jax
import jax.numpy as jnp
from jax.experimental import pallas as pl


def kernel(x, edge_index_follows, edge_index_likes, edge_index_views, W_follows, W_likes, W_views):
    raise NotImplementedError("write your pallas kernel here")


# speedup vs baseline: 7.0316x; 7.0316x over previous
import jax
import jax.numpy as jnp
from jax import lax
from jax.experimental import pallas as pl
from jax.experimental.pallas import tpu as pltpu
from jax.experimental.pallas import tpu_sc as plsc

_N = 10000
_D = 128
_E = 320000
_NC = 2
_NS = 16
_NW = _NC * _NS
_C = 125
_RB = 8
_EPW = _E // _NW
_BLKS = _EPW // (_C * _RB)
_RPS = 640
_NP = _NS * _RPS
_NCHUNK = _E // _C


def _mm_body(x_ref, w0_ref, w1_ref, w2_ref, o0_ref, o1_ref, o2_ref):
    xv = x_ref[...]
    o0_ref[...] = jnp.dot(xv, w0_ref[...], preferred_element_type=jnp.float32)
    o1_ref[...] = jnp.dot(xv, w1_ref[...], preferred_element_type=jnp.float32)
    o2_ref[...] = jnp.dot(xv, w2_ref[...], preferred_element_type=jnp.float32)


def _sc_body(wh0, wh1, wh2, s0, d0, s1, d1, s2, d2, zrows, part,
             idx_s, idx_d, rows, acc, sem):
    c = lax.axis_index("c")
    s = lax.axis_index("s")
    w = s * _NC + c
    pltpu.sync_copy(zrows, acc.at[pl.ds(s * _RPS, _RPS)])
    plsc.subcore_barrier()
    for wh, si, di in ((wh0, s0, d0), (wh1, s1, d1), (wh2, s2, d2)):
        @pl.loop(0, _BLKS)
        def _blk(b):
            r0 = w * (_BLKS * _RB) + b * _RB
            pltpu.sync_copy(si.at[pl.ds(r0, _RB)], idx_s)
            pltpu.sync_copy(di.at[pl.ds(r0, _RB)], idx_d)
            for j in range(_RB):
                pltpu.async_copy(wh.at[idx_s.at[j]], rows, sem).wait()
                pltpu.sync_copy(rows, acc.at[idx_d.at[j]], add=True)
    plsc.subcore_barrier()
    pltpu.sync_copy(acc.at[pl.ds(s * _RPS, _RPS)],
                    part.at[pl.ds(c * _NP + s * _RPS, _RPS)])


_sc_call = pl.kernel(
    _sc_body,
    out_type=jax.ShapeDtypeStruct((_NC * _NP, _D), jnp.float32),
    mesh=plsc.VectorSubcoreMesh(core_axis_name="c", subcore_axis_name="s",
                                num_cores=_NC, num_subcores=_NS),
    scratch_types=[
        pltpu.VMEM((_RB, _C), jnp.int32),
        pltpu.VMEM((_RB, _C), jnp.int32),
        pltpu.VMEM((_C, _D), jnp.float32),
        pltpu.VMEM_SHARED((_NP, _D), jnp.float32),
        pltpu.SemaphoreType.DMA,
    ],
)


def kernel(x, edge_index_follows, edge_index_likes, edge_index_views,
           W_follows, W_likes, W_views):
    wh0, wh1, wh2 = pl.pallas_call(
        _mm_body,
        out_shape=[jax.ShapeDtypeStruct((_N, _D), jnp.float32)] * 3,
    )(x, W_follows, W_likes, W_views)
    idx = []
    for ei in (edge_index_follows, edge_index_likes, edge_index_views):
        e32 = ei.astype(jnp.int32)
        idx.append(e32[0].reshape(_NCHUNK, _C))
        idx.append(e32[1].reshape(_NCHUNK, _C))
    zrows = jnp.zeros((_RPS, _D), jnp.float32)
    part = _sc_call(wh0, wh1, wh2, *idx, zrows)
    return part[:_N] + part[_NP:_NP + _N]


# speedup vs baseline: 10.6472x; 1.5142x over previous
import jax
import jax.numpy as jnp
from jax import lax
from jax.experimental import pallas as pl
from jax.experimental.pallas import tpu as pltpu
from jax.experimental.pallas import tpu_sc as plsc

_N = 10000
_D = 128
_E = 320000
_NC = 2
_NS = 16
_NW = _NC * _NS
_C = 125
_RB = 8
_EPW = _E // _NW
_BLKS = _EPW // (_C * _RB)
_RPS = 640
_NP = _NS * _RPS
_NCHUNK = _E // _C


def _mm_body(x_ref, w0_ref, w1_ref, w2_ref, o0_ref, o1_ref, o2_ref):
    xv = x_ref[...]
    o0_ref[...] = jnp.dot(xv, w0_ref[...], preferred_element_type=jnp.float32)
    o1_ref[...] = jnp.dot(xv, w1_ref[...], preferred_element_type=jnp.float32)
    o2_ref[...] = jnp.dot(xv, w2_ref[...], preferred_element_type=jnp.float32)


_CPW = _BLKS * _RB


def _sc_body(wh0, wh1, wh2, s0, d0, s1, d1, s2, d2, zrows, part,
             idx_s, idx_d, rows, acc, sem):
    c = lax.axis_index("c")
    s = lax.axis_index("s")
    w = s * _NC + c
    pltpu.sync_copy(zrows, acc.at[pl.ds(s * _RPS, _RPS)])
    plsc.subcore_barrier()
    for wh, si, di in ((wh0, s0, d0), (wh1, s1, d1), (wh2, s2, d2)):
        base = w * _CPW

        def _gather(pg, row, buf):
            return pltpu.make_async_copy(wh.at[idx_s.at[pg, row]],
                                         rows.at[buf], sem)

        pltpu.sync_copy(si.at[pl.ds(base, _RB)], idx_s.at[0])
        pltpu.sync_copy(di.at[pl.ds(base, _RB)], idx_d.at[0])
        _gather(0, 0, 0).start()

        @pl.loop(0, _CPW)
        def _chunk(k):
            kn = k + 1

            @pl.when(kn < _CPW)
            def _fire():
                pgn = lax.rem(lax.div(kn, _RB), 2)
                rown = lax.rem(kn, _RB)

                @pl.when(rown == 0)
                def _page():
                    off = pl.multiple_of(base + lax.div(kn, _RB) * _RB, _RB)
                    pltpu.sync_copy(si.at[pl.ds(off, _RB)], idx_s.at[pgn])
                    pltpu.sync_copy(di.at[pl.ds(off, _RB)], idx_d.at[pgn])

                _gather(pgn, rown, lax.rem(kn, 2)).start()

            pg = lax.rem(lax.div(k, _RB), 2)
            row = lax.rem(k, _RB)
            buf = lax.rem(k, 2)
            _gather(pg, row, buf).wait()
            pltpu.sync_copy(rows.at[buf], acc.at[idx_d.at[pg, row]], add=True)
    plsc.subcore_barrier()
    pltpu.sync_copy(acc.at[pl.ds(s * _RPS, _RPS)],
                    part.at[pl.ds(c * _NP + s * _RPS, _RPS)])


_sc_call = pl.kernel(
    _sc_body,
    out_type=jax.ShapeDtypeStruct((_NC * _NP, _D), jnp.float32),
    mesh=plsc.VectorSubcoreMesh(core_axis_name="c", subcore_axis_name="s",
                                num_cores=_NC, num_subcores=_NS),
    scratch_types=[
        pltpu.VMEM((2, _RB, _C), jnp.int32),
        pltpu.VMEM((2, _RB, _C), jnp.int32),
        pltpu.VMEM((2, _C, _D), jnp.float32),
        pltpu.VMEM_SHARED((_NP, _D), jnp.float32),
        pltpu.SemaphoreType.DMA,
    ],
)


def kernel(x, edge_index_follows, edge_index_likes, edge_index_views,
           W_follows, W_likes, W_views):
    wh0, wh1, wh2 = pl.pallas_call(
        _mm_body,
        out_shape=[jax.ShapeDtypeStruct((_N, _D), jnp.float32)] * 3,
    )(x, W_follows, W_likes, W_views)
    idx = []
    for ei in (edge_index_follows, edge_index_likes, edge_index_views):
        e32 = ei.astype(jnp.int32)
        idx.append(e32[0].reshape(_NCHUNK, _C))
        idx.append(e32[1].reshape(_NCHUNK, _C))
    zrows = jnp.zeros((_RPS, _D), jnp.float32)
    part = _sc_call(wh0, wh1, wh2, *idx, zrows)
    return part[:_N] + part[_NP:_NP + _N]
